# Initial kernel scaffold; baseline (speedup 1.0000x reference)
import jax
import jax.numpy as jnp
from jax.experimental import pallas as pl


def kernel(x, weight):
    raise NotImplementedError("write your pallas kernel here")


# speedup vs baseline: 1.4028x; 1.4028x over previous
import functools

import jax
import jax.numpy as jnp
from jax import lax
from jax.experimental import pallas as pl
from jax.experimental.pallas import tpu as pltpu
from jax.experimental.pallas import tpu_sc as plsc

NUM_EMB = 1000000
D = 32
BATCH = 16384
HIST = 20
TOTAL = BATCH * HIST
NC = 2
NS = 16
NW = NC * NS
PER_W = TOTAL // NW
CHUNK = 128
NCHUNK = PER_W // CHUNK


def _emb_body(idx_hbm, table_hbm, out_hbm, idx_v, rows_v, sem):
    wid = lax.axis_index("s") * NC + lax.axis_index("c")
    pltpu.sync_copy(idx_hbm.at[wid], idx_v)

    def body(c, _):
        pltpu.async_copy(table_hbm.at[idx_v.at[c]], rows_v, sem).wait()
        pltpu.sync_copy(rows_v, out_hbm.at[wid, c])
        return 0

    lax.fori_loop(0, NCHUNK, body, 0)


@jax.jit
def _emb(x3, weight):
    mesh = plsc.VectorSubcoreMesh(core_axis_name="c", subcore_axis_name="s")
    f = pl.kernel(
        _emb_body,
        mesh=mesh,
        out_type=jax.ShapeDtypeStruct((NW, NCHUNK, CHUNK, D), jnp.float32),
        scratch_types=[
            pltpu.VMEM((NCHUNK, CHUNK), jnp.int32),
            pltpu.VMEM((CHUNK, D), jnp.float32),
            pltpu.SemaphoreType.DMA,
        ],
        compiler_params=pltpu.CompilerParams(use_tc_tiling_on_sc=False),
    )
    return f(x3, weight)


def kernel(x, weight):
    x3 = x.reshape(NW, NCHUNK, CHUNK)
    out = _emb(x3, weight)
    return out.reshape(BATCH, HIST, D)


# speedup vs baseline: 1.5139x; 1.0792x over previous
import functools

import jax
import jax.numpy as jnp
from jax import lax
from jax.experimental import pallas as pl
from jax.experimental.pallas import tpu as pltpu
from jax.experimental.pallas import tpu_sc as plsc

NUM_EMB = 1000000
D = 32
BATCH = 16384
HIST = 20
TOTAL = BATCH * HIST
NC = 2
NS = 16
NW = NC * NS
PER_W = TOTAL // NW
CHUNK = 128
NCHUNK = PER_W // CHUNK
K = 8
NGROUP = NCHUNK // K


def _emb_body(idx_hbm, table_hbm, out_hbm, idx_v, buf_a, buf_b, sem_a, sem_b):
    wid = lax.axis_index("s") * NC + lax.axis_index("c")
    pltpu.sync_copy(idx_hbm.at[wid], idx_v)

    def fire(g, buf, sem):
        for j in range(K):
            pltpu.async_copy(table_hbm.at[idx_v.at[g * K + j]], buf.at[j], sem)

    def drain(buf, sem):
        for j in range(K):
            pltpu.make_async_copy(table_hbm.at[pl.ds(0, CHUNK)], buf.at[j], sem).wait()

    fire(0, buf_a, sem_a)

    def body(p, _):
        ga = 2 * p
        fire(ga + 1, buf_b, sem_b)
        drain(buf_a, sem_a)
        pltpu.sync_copy(buf_a, out_hbm.at[wid, pl.ds(ga * K, K)])

        @pl.when(ga + 2 < NGROUP)
        def _():
            fire(ga + 2, buf_a, sem_a)

        drain(buf_b, sem_b)
        pltpu.sync_copy(buf_b, out_hbm.at[wid, pl.ds((ga + 1) * K, K)])
        return 0

    lax.fori_loop(0, NGROUP // 2, body, 0)


@jax.jit
def _emb(x3, weight):
    mesh = plsc.VectorSubcoreMesh(core_axis_name="c", subcore_axis_name="s")
    f = pl.kernel(
        _emb_body,
        mesh=mesh,
        out_type=jax.ShapeDtypeStruct((NW, NCHUNK, CHUNK, D), jnp.float32),
        scratch_types=[
            pltpu.VMEM((NCHUNK, CHUNK), jnp.int32),
            pltpu.VMEM((K, CHUNK, D), jnp.float32),
            pltpu.VMEM((K, CHUNK, D), jnp.float32),
            pltpu.SemaphoreType.DMA,
            pltpu.SemaphoreType.DMA,
        ],
        compiler_params=pltpu.CompilerParams(use_tc_tiling_on_sc=False),
    )
    return f(x3, weight)


def kernel(x, weight):
    x3 = x.reshape(NW, NCHUNK, CHUNK)
    out = _emb(x3, weight)
    return out.reshape(BATCH, HIST, D)
